# baseline (speedup 1.0000x reference)
import functools

import jax
import jax.numpy as jnp
from jax.experimental import pallas as pl


def _tc_body(a_ref, d_ref, w_ref, b_ref, o_ref, *, last):
  acc = a_ref[...]
  deg = d_ref[...]
  m = acc / jnp.maximum(deg, 1.0)
  h = jnp.dot(m, w_ref[...], preferred_element_type=jnp.float32) + b_ref[...]
  if last:
    mx = jnp.max(h, axis=-1, keepdims=True)
    lse = jnp.log(jnp.sum(jnp.exp(h - mx), axis=-1, keepdims=True)) + mx
    o_ref[...] = h - lse
  else:
    o_ref[...] = jnp.maximum(h, 0.0)


def _tc_dense(acc, deg, W, b, last):
  return pl.pallas_call(
      functools.partial(_tc_body, last=last),
      out_shape=jax.ShapeDtypeStruct((acc.shape[0], W.shape[1]), jnp.float32),
  )(acc, deg.reshape(-1, 1), W, b.reshape(1, -1))


def _agg(x_all, n_tgt, adj):
  msgs = jnp.take(x_all, adj[0], axis=0)
  agg = jax.ops.segment_sum(msgs, adj[1], num_segments=n_tgt)
  deg = jax.ops.segment_sum(jnp.ones(adj.shape[1], x_all.dtype), adj[1],
                            num_segments=n_tgt)
  return agg, deg


def kernel(x, adj0, adj1, W0, b0, W1, b1):
  acc0, deg0 = _agg(x, 4096, adj0)
  h1 = _tc_dense(acc0, deg0, W0, b0, last=False)
  acc1, deg1 = _agg(h1, 1024, adj1)
  return _tc_dense(acc1, deg1, W1, b1, last=True)
